# baseline (speedup 1.0000x reference)
import math

import jax
import jax.numpy as jnp
from jax import lax
from jax.experimental import pallas as pl
from jax.experimental.pallas import tpu as pltpu
from jax.experimental.pallas import tpu_sc as plsc

_V = 1000000
_VP = _V // 2
_D = 64
_B = 4096
_H = 200
_NC = 2
_NS = 16
_NW = _NC * _NS
_BB = _B // _NW
_NG = 4
_NO = 2
_SCALE = math.sqrt(_D)
_L = 16


def _emb_body(xt_hbm, lut2_hbm, out_hbm, idx_all, pidx, poff, gbuf, obuf,
              gsem, ssem):
    cid = lax.axis_index("c")
    sid = lax.axis_index("s")
    wid = sid * _NC + cid
    b0 = wid * _BB

    pltpu.sync_copy(xt_hbm.at[:, pl.ds(b0, _BB)], idx_all)

    lane = lax.iota(jnp.int32, _L)

    def prep_and_gather(s, h):
        for j in range(_BB // _L):
            sl = pl.ds(j * _L, _L)
            v = idx_all[h, sl]
            pidx[s, sl] = v >> 1
            poff[s, sl] = (v & 1) * _D
        pltpu.async_copy(lut2_hbm.at[pidx.at[s]], gbuf.at[s], gsem.at[s])

    def wait_gather(s):
        pltpu.make_async_copy(
            lut2_hbm.at[pidx.at[s]], gbuf.at[s], gsem.at[s]
        ).wait()

    def start_store(so, h):
        pltpu.async_copy(
            obuf.at[so], out_hbm.at[h, :, pl.ds(b0, _BB)], ssem.at[so]
        )

    def wait_store(so):
        pltpu.make_async_copy(
            obuf.at[so], out_hbm.at[0, :, pl.ds(b0, _BB)], ssem.at[so]
        ).wait()

    for s in range(_NG):
        prep_and_gather(s, s)

    def outer(g, carry):
        for s in range(_NG):
            h = g * _NG + s
            so = s % _NO
            wait_gather(s)

            @pl.when(h >= _NO)
            def _():
                wait_store(so)

            src = gbuf.at[s]
            dst = obuf.at[so]
            rows = [lane + (jb * _L) for jb in range(_BB // _L)]
            cols = [poff[s, pl.ds(jb * _L, _L)] for jb in range(_BB // _L)]

            @plsc.parallel_loop(0, _D, step=1, unroll=4)
            def _(d):
                for jb in range(_BB // _L):
                    vals = plsc.load_gather(src, [rows[jb], cols[jb] + d])
                    dst[d, pl.ds(jb * _L, _L)] = vals * _SCALE

            start_store(so, h)

            @pl.when(g < (_H // _NG) - 1)
            def _():
                prep_and_gather(s, h + _NG)
        return carry

    lax.fori_loop(0, _H // _NG, outer, 0)

    for so in range(_NO):
        wait_store(so)


def kernel(x, lut):
    xt = x.T.astype(jnp.int32)
    lut2 = lut.reshape(_VP, 2 * _D)
    mesh = plsc.VectorSubcoreMesh(
        core_axis_name="c", subcore_axis_name="s",
        num_cores=_NC, num_subcores=_NS,
    )
    out_t = pl.kernel(
        _emb_body,
        out_type=jax.ShapeDtypeStruct((_H, _D, _B), jnp.float32),
        mesh=mesh,
        compiler_params=pltpu.CompilerParams(
            use_tc_tiling_on_sc=True, needs_layout_passes=False,
        ),
        scratch_types=[
            pltpu.VMEM((_H, _BB), jnp.int32),
            pltpu.VMEM((_NG, _BB), jnp.int32),
            pltpu.VMEM((_NG, _BB), jnp.int32),
            pltpu.VMEM((_NG, _BB, 2 * _D), jnp.float32),
            pltpu.VMEM((_NO, _D, _BB), jnp.float32),
            pltpu.SemaphoreType.DMA((_NG,)),
            pltpu.SemaphoreType.DMA((_NO,)),
        ],
    )(xt, lut2)
    return out_t.transpose(2, 0, 1)


# speedup vs baseline: 1.9602x; 1.9602x over previous
import math

import jax
import jax.numpy as jnp
from jax import lax
from jax.experimental import pallas as pl
from jax.experimental.pallas import tpu as pltpu
from jax.experimental.pallas import tpu_sc as plsc

_V = 1000000
_F = 499968
_VP = _F + 64
_NFULL = _F // 128
_TAIL = _V - _F - 128
_D = 64
_B = 4096
_H = 200
_NC = 2
_NS = 16
_NW = _NC * _NS
_BB = _B // _NW
_NG = 4
_NO = 2
_SCALE = math.sqrt(_D)
_L = 16


def _wid():
    return lax.axis_index("s") * _NC + lax.axis_index("c")


def _fold_body(lutt_hbm, lut2_hbm, islab, oslab, tailbuf, isem, osem):
    wid = _wid()
    lane = lax.iota(jnp.int32, _L)

    def start_in(s, blk):
        c0 = blk * 128
        pltpu.async_copy(lutt_hbm.at[:, pl.ds(c0, 128)], islab.at[s, 0],
                         isem.at[s])
        pltpu.async_copy(lutt_hbm.at[:, pl.ds(_F + c0, 128)], islab.at[s, 1],
                         isem.at[s])

    def wait_in(s):
        pltpu.make_async_copy(lutt_hbm.at[:, pl.ds(0, 128)], islab.at[s, 0],
                              isem.at[s]).wait()
        pltpu.make_async_copy(lutt_hbm.at[:, pl.ds(0, 128)], islab.at[s, 1],
                              isem.at[s]).wait()

    def start_out(s, blk):
        pltpu.async_copy(oslab.at[s], lut2_hbm.at[pl.ds(blk * 128, 128)],
                         osem.at[s])

    def wait_out(s):
        pltpu.make_async_copy(oslab.at[s], lut2_hbm.at[pl.ds(0, 128)],
                              osem.at[s]).wait()

    for s in range(2):
        blk = wid + _NW * s

        @pl.when(blk < _NFULL)
        def _():
            start_in(s, blk)

    def body(i, carry):
        for s in range(2):
            blk = wid + _NW * (2 * i + s)
            nblk = wid + _NW * (2 * i + s + 2)

            @pl.when(blk < _NFULL)
            def _():
                wait_in(s)

                @pl.when(2 * i + s >= 2)
                def _():
                    wait_out(s)

                dst = oslab.at[s]
                for h in range(2):
                    src = islab.at[s, h]

                    @plsc.parallel_loop(0, _L, step=1, unroll=4)
                    def _(d):
                        diag = (lane + d) & (_L - 1)
                        for eb in range(4):
                            for qb in range(8):
                                vals = plsc.load_gather(
                                    src,
                                    [diag + eb * _L, lane + qb * _L])
                                plsc.store_scatter(
                                    dst,
                                    [lane + qb * _L,
                                     diag + eb * _L + h * _D],
                                    vals * _SCALE)

                start_out(s, blk)

                @pl.when(nblk < _NFULL)
                def _():
                    start_in(s, nblk)
        return carry

    lax.fori_loop(0, 62, body, 0)

    for s in range(2):
        wait_out(s)

    @pl.when(wid == 0)
    def _():
        pltpu.sync_copy(lutt_hbm.at[:, pl.ds(_V - _D, _D)], tailbuf)
        dst = oslab.at[0]

        @plsc.parallel_loop(0, _L, step=1, unroll=4)
        def _(d):
            diag = (lane + d) & (_L - 1)
            for eb in range(4):
                for qb in range(4):
                    vals = plsc.load_gather(
                        tailbuf, [diag + eb * _L, lane + qb * _L])
                    plsc.store_scatter(
                        dst, [lane + qb * _L, diag + eb * _L + _D],
                        vals * _SCALE)
        pltpu.sync_copy(dst.at[pl.ds(0, _D)], lut2_hbm.at[pl.ds(_F, _D)])


def _gather_body(xt_hbm, lut2_hbm, out_hbm, idx_all, pidx, poff, gbuf, obuf,
                 gsem, ssem):
    wid = _wid()
    b0 = wid * _BB
    lane = lax.iota(jnp.int32, _L)

    pltpu.sync_copy(xt_hbm.at[:, pl.ds(b0, _BB)], idx_all)

    def prep_and_gather(s, h):
        for j in range(_BB // _L):
            sl = pl.ds(j * _L, _L)
            v = idx_all[h, sl]
            big = v >= _F
            pidx[s, sl] = jnp.where(big, v - _F, v)
            poff[s, sl] = jnp.where(big, _D, 0)
        pltpu.async_copy(lut2_hbm.at[pidx.at[s]], gbuf.at[s], gsem.at[s])

    def wait_gather(s):
        pltpu.make_async_copy(lut2_hbm.at[pidx.at[s]], gbuf.at[s],
                              gsem.at[s]).wait()

    def start_store(so, h):
        pltpu.async_copy(obuf.at[so], out_hbm.at[h, :, pl.ds(b0, _BB)],
                         ssem.at[so])

    def wait_store(so):
        pltpu.make_async_copy(obuf.at[so], out_hbm.at[0, :, pl.ds(b0, _BB)],
                              ssem.at[so]).wait()

    for s in range(_NG):
        prep_and_gather(s, s)

    def outer(g, carry):
        for s in range(_NG):
            h = g * _NG + s
            so = s % _NO
            wait_gather(s)

            @pl.when(h >= _NO)
            def _():
                wait_store(so)

            src = gbuf.at[s]
            dst = obuf.at[so]
            offs = [poff[s, pl.ds(jb * _L, _L)] for jb in range(_BB // _L)]

            @plsc.parallel_loop(0, _L, step=1, unroll=4)
            def _(d):
                diag = (lane + d) & (_L - 1)
                for db in range(_D // _L):
                    for jb in range(_BB // _L):
                        vals = plsc.load_gather(
                            src,
                            [lane + jb * _L, offs[jb] + (diag + db * _L)])
                        plsc.store_scatter(
                            dst, [diag + db * _L, lane + jb * _L], vals)

            start_store(so, h)

            @pl.when(g < (_H // _NG) - 1)
            def _():
                prep_and_gather(s, h + _NG)
        return carry

    lax.fori_loop(0, _H // _NG, outer, 0)

    for so in range(_NO):
        wait_store(so)


def kernel(x, lut):
    xt = x.T.astype(jnp.int32)
    lutt = lut.T
    mesh = plsc.VectorSubcoreMesh(
        core_axis_name="c", subcore_axis_name="s",
        num_cores=_NC, num_subcores=_NS,
    )
    params = pltpu.CompilerParams(
        use_tc_tiling_on_sc=True, needs_layout_passes=False,
    )
    lut2 = pl.kernel(
        _fold_body,
        out_type=jax.ShapeDtypeStruct((_VP, 2 * _D), jnp.float32),
        mesh=mesh,
        compiler_params=params,
        scratch_types=[
            pltpu.VMEM((2, 2, _D, 128), jnp.float32),
            pltpu.VMEM((2, 128, 2 * _D), jnp.float32),
            pltpu.VMEM((_D, _D), jnp.float32),
            pltpu.SemaphoreType.DMA((2,)),
            pltpu.SemaphoreType.DMA((2,)),
        ],
    )(lutt)
    out_t = pl.kernel(
        _gather_body,
        out_type=jax.ShapeDtypeStruct((_H, _D, _B), jnp.float32),
        mesh=mesh,
        compiler_params=params,
        scratch_types=[
            pltpu.VMEM((_H, _BB), jnp.int32),
            pltpu.VMEM((_NG, _BB), jnp.int32),
            pltpu.VMEM((_NG, _BB), jnp.int32),
            pltpu.VMEM((_NG, _BB, 2 * _D), jnp.float32),
            pltpu.VMEM((_NO, _D, _BB), jnp.float32),
            pltpu.SemaphoreType.DMA((_NG,)),
            pltpu.SemaphoreType.DMA((_NO,)),
        ],
    )(xt, lut2)
    return out_t.transpose(2, 0, 1)
